# Initial kernel scaffold; baseline (speedup 1.0000x reference)
import functools

import jax
import jax.numpy as jnp
from jax import lax
from jax.experimental import pallas as pl
from jax.experimental.pallas import tpu as pltpu
from jax.experimental.pallas import tpu_sc as plsc

B_, S_, D_, H_, E_ = 2, 2048, 1024, 2048, 8
T_ = B_ * S_
C_ = 640
NSLOT = E_ * C_
NC, NS = 2, 16
NW = NC * NS
TPW = T_ // NW
RBUF = NSLOT + NW

TB = 512
HT = 256


def _gate_body(x_ref, wg_ref, slot_ref, gmul_ref, dst_ref, carry_ref):
    i = pl.program_id(0)
    x = x_ref[...]
    logits = jnp.dot(x, wg_ref[...], preferred_element_type=jnp.float32)
    lmax = jnp.max(logits, axis=1, keepdims=True)
    gval = 1.0 / jnp.sum(jnp.exp(logits - lmax), axis=1, keepdims=True)
    eids = lax.broadcasted_iota(jnp.int32, logits.shape, 1)
    eidx = jnp.min(jnp.where(logits >= lmax, eids, E_), axis=1, keepdims=True)

    @pl.when(i == 0)
    def _():
        carry_ref[...] = jnp.zeros_like(carry_ref)

    onehot = (eids == eidx).astype(jnp.float32)
    r = lax.broadcasted_iota(jnp.int32, (TB, TB), 0)
    c = lax.broadcasted_iota(jnp.int32, (TB, TB), 1)
    tri = (r >= c).astype(jnp.float32)
    cnt = jnp.dot(tri, onehot, preferred_element_type=jnp.float32)
    cnt = cnt + carry_ref[...]
    carry_ref[...] = cnt[TB - 1:TB, :]
    pos = jnp.sum(onehot * cnt, axis=1, keepdims=True).astype(jnp.int32) - 1
    valid = pos < C_
    slot = eidx * C_ + jnp.minimum(pos, C_ - 1)
    trow = lax.broadcasted_iota(jnp.int32, (TB, 1), 0) + i * TB
    dump = NSLOT + trow // TPW
    slot_ref[...] = slot
    gmul_ref[...] = jnp.where(valid, gval, 0.0)
    dst_ref[...] = jnp.where(valid, slot, dump)


def _gating(xf, wg):
    return pl.pallas_call(
        _gate_body,
        grid=(T_ // TB,),
        in_specs=[
            pl.BlockSpec((TB, D_), lambda i: (i, 0)),
            pl.BlockSpec((D_, E_), lambda i: (0, 0)),
        ],
        out_specs=[
            pl.BlockSpec((TB, 1), lambda i: (i, 0)),
            pl.BlockSpec((TB, 1), lambda i: (i, 0)),
            pl.BlockSpec((TB, 1), lambda i: (i, 0)),
        ],
        out_shape=[
            jax.ShapeDtypeStruct((T_, 1), jnp.int32),
            jax.ShapeDtypeStruct((T_, 1), jnp.float32),
            jax.ShapeDtypeStruct((T_, 1), jnp.int32),
        ],
        scratch_shapes=[pltpu.VMEM((1, E_), jnp.float32)],
    )(xf, wg)


def _scatter_body(dst_hbm, x_hbm, bufx_hbm, dst_v, rows_v, sem):
    wid = lax.axis_index("c") * NS + lax.axis_index("s")
    base = wid * TPW
    pltpu.sync_copy(dst_hbm.at[pl.ds(base, TPW)], dst_v)
    for v in range(TPW // 16):
        idx = dst_v[pl.ds(v * 16, 16)]
        pltpu.sync_copy(x_hbm.at[pl.ds(base + v * 16, 16)], rows_v)
        pltpu.async_copy(rows_v, bufx_hbm.at[idx], sem).wait()


def _scatter(dst, xf):
    mesh = plsc.VectorSubcoreMesh(core_axis_name="c", subcore_axis_name="s")
    return pl.kernel(
        _scatter_body,
        out_type=jax.ShapeDtypeStruct((RBUF, D_), jnp.float32),
        mesh=mesh,
        scratch_types=[
            pltpu.VMEM((TPW,), jnp.int32),
            pltpu.VMEM((16, D_), jnp.float32),
            pltpu.SemaphoreType.DMA,
        ],
    )(dst, xf)


def _ffn_body(bufx_ref, w1_ref, b1_ref, w2_ref, b2_ref, out_ref):
    ht = pl.program_id(1)
    xb = bufx_ref[...]
    hblk = jnp.dot(xb, w1_ref[0], preferred_element_type=jnp.float32)
    hblk = jnp.maximum(hblk + b1_ref[0], 0.0)
    w2s = jnp.sum(w2_ref[0], axis=1, keepdims=True)
    part = jnp.dot(hblk, w2s, preferred_element_type=jnp.float32)

    @pl.when(ht == 0)
    def _():
        out_ref[...] = part + jnp.sum(b2_ref[0])

    @pl.when(ht != 0)
    def _():
        out_ref[...] += part


def _ffn(bufx, w1, b1r, w2, b2r):
    return pl.pallas_call(
        _ffn_body,
        grid=(E_, H_ // HT),
        in_specs=[
            pl.BlockSpec((C_, D_), lambda e, h: (e, 0)),
            pl.BlockSpec((1, D_, HT), lambda e, h: (e, 0, h)),
            pl.BlockSpec((1, 1, HT), lambda e, h: (e, 0, h)),
            pl.BlockSpec((1, HT, D_), lambda e, h: (e, h, 0)),
            pl.BlockSpec((1, 1, D_), lambda e, h: (e, 0, 0)),
        ],
        out_specs=pl.BlockSpec((C_, 1), lambda e, h: (e, 0)),
        out_shape=jax.ShapeDtypeStruct((NSLOT, 1), jnp.float32),
    )(bufx, w1, b1r, w2, b2r)


def _combine_body(slot_hbm, gmul_hbm, s1_hbm, z_hbm, sl_v, gm_v, s1_v, z_v):
    wid = lax.axis_index("c") * NS + lax.axis_index("s")
    base = wid * TPW
    pltpu.sync_copy(slot_hbm.at[pl.ds(base, TPW)], sl_v)
    pltpu.sync_copy(gmul_hbm.at[pl.ds(base, TPW)], gm_v)
    pltpu.sync_copy(s1_hbm, s1_v)
    for v in range(TPW // 16):
        idx = sl_v[pl.ds(v * 16, 16)]
        vals = plsc.load_gather(s1_v, [idx])
        z_v[pl.ds(v * 16, 16)] = vals * gm_v[pl.ds(v * 16, 16)]
    pltpu.sync_copy(z_v, z_hbm.at[pl.ds(base, TPW)])


def _combine(slot, gmul, s1):
    mesh = plsc.VectorSubcoreMesh(core_axis_name="c", subcore_axis_name="s")
    return pl.kernel(
        _combine_body,
        out_type=jax.ShapeDtypeStruct((T_,), jnp.float32),
        mesh=mesh,
        scratch_types=[
            pltpu.VMEM((TPW,), jnp.int32),
            pltpu.VMEM((TPW,), jnp.float32),
            pltpu.VMEM((NSLOT,), jnp.float32),
            pltpu.VMEM((TPW,), jnp.float32),
        ],
    )(slot, gmul, s1)


def _lsm_body(z_ref, out_ref):
    z = z_ref[...]
    m = jnp.max(z, axis=1, keepdims=True)
    out_ref[...] = z - m - jnp.log(jnp.sum(jnp.exp(z - m), axis=1, keepdims=True))


def _lsm(z2):
    return pl.pallas_call(
        _lsm_body,
        out_shape=jax.ShapeDtypeStruct((B_, S_), jnp.float32),
    )(z2)


def kernel(input, Wg, W1, b1, W2, b2):
    xf = input.reshape(T_, D_)
    slot, gmul, dst = _gating(xf, Wg)
    bufx = _scatter(dst.reshape(T_), xf)
    s1 = _ffn(bufx, W1, b1.reshape(E_, 1, H_), W2, b2.reshape(E_, 1, D_))
    z = _combine(slot.reshape(T_), gmul.reshape(T_), s1.reshape(NSLOT))
    return _lsm(z.reshape(B_, S_))


# speedup vs baseline: 1.2439x; 1.2439x over previous
import functools

import jax
import jax.numpy as jnp
from jax import lax
from jax.experimental import pallas as pl
from jax.experimental.pallas import tpu as pltpu
from jax.experimental.pallas import tpu_sc as plsc

B_, S_, D_, H_, E_ = 2, 2048, 1024, 2048, 8
T_ = B_ * S_
C_ = 640
NSLOT = E_ * C_
NC, NS = 2, 16
NW = NC * NS
TPW = T_ // NW
RBUF = NSLOT + NW

TB = 512
HT = 256


def _gate_body(x_ref, wg_ref, slot_ref, gmul_ref, dst_ref, carry_ref):
    i = pl.program_id(0)
    x = x_ref[...]
    logits = jnp.dot(x, wg_ref[...], preferred_element_type=jnp.float32)
    lmax = jnp.max(logits, axis=1, keepdims=True)
    gval = 1.0 / jnp.sum(jnp.exp(logits - lmax), axis=1, keepdims=True)
    eids = lax.broadcasted_iota(jnp.int32, logits.shape, 1)
    eidx = jnp.min(jnp.where(logits >= lmax, eids, E_), axis=1, keepdims=True)

    @pl.when(i == 0)
    def _():
        carry_ref[...] = jnp.zeros_like(carry_ref)

    onehot = (eids == eidx).astype(jnp.float32)
    r = lax.broadcasted_iota(jnp.int32, (TB, TB), 0)
    c = lax.broadcasted_iota(jnp.int32, (TB, TB), 1)
    tri = (r >= c).astype(jnp.float32)
    cnt = jnp.dot(tri, onehot, preferred_element_type=jnp.float32)
    cnt = cnt + carry_ref[...]
    carry_ref[...] = cnt[TB - 1:TB, :]
    pos = jnp.sum(onehot * cnt, axis=1, keepdims=True).astype(jnp.int32) - 1
    valid = pos < C_
    slot = eidx * C_ + jnp.minimum(pos, C_ - 1)
    trow = lax.broadcasted_iota(jnp.int32, (TB, 1), 0) + i * TB
    dump = NSLOT + trow // TPW
    slot_ref[...] = slot
    gmul_ref[...] = jnp.where(valid, gval, 0.0)
    dst_ref[...] = jnp.where(valid, slot, dump)


def _gating(xf, wg):
    return pl.pallas_call(
        _gate_body,
        grid=(T_ // TB,),
        in_specs=[
            pl.BlockSpec((TB, D_), lambda i: (i, 0)),
            pl.BlockSpec((D_, E_), lambda i: (0, 0)),
        ],
        out_specs=[
            pl.BlockSpec((TB, 1), lambda i: (i, 0)),
            pl.BlockSpec((TB, 1), lambda i: (i, 0)),
            pl.BlockSpec((TB, 1), lambda i: (i, 0)),
        ],
        out_shape=[
            jax.ShapeDtypeStruct((T_, 1), jnp.int32),
            jax.ShapeDtypeStruct((T_, 1), jnp.float32),
            jax.ShapeDtypeStruct((T_, 1), jnp.int32),
        ],
        scratch_shapes=[pltpu.VMEM((1, E_), jnp.float32)],
    )(xf, wg)


def _scatter_body(dst_hbm, x_hbm, bufx_hbm, dst_v, rows_v, sem):
    wid = lax.axis_index("c") * NS + lax.axis_index("s")
    base = wid * TPW
    pltpu.sync_copy(dst_hbm.at[pl.ds(base, TPW)], dst_v)
    for v in range(TPW // 16):
        idx = dst_v[pl.ds(v * 16, 16)]
        pltpu.sync_copy(x_hbm.at[pl.ds(base + v * 16, 16)], rows_v)
        pltpu.async_copy(rows_v, bufx_hbm.at[idx], sem).wait()


def _scatter(dst, xf):
    mesh = plsc.VectorSubcoreMesh(core_axis_name="c", subcore_axis_name="s")
    return pl.kernel(
        _scatter_body,
        out_type=jax.ShapeDtypeStruct((RBUF, D_), jnp.float32),
        mesh=mesh,
        scratch_types=[
            pltpu.VMEM((TPW,), jnp.int32),
            pltpu.VMEM((16, D_), jnp.float32),
            pltpu.SemaphoreType.DMA,
        ],
    )(dst, xf)


def _ffn_body(bufx_ref, w1_ref, b1_ref, w2_ref, b2_ref, out_ref):
    ht = pl.program_id(1)
    xb = bufx_ref[...]
    hblk = jnp.dot(xb, w1_ref[0], preferred_element_type=jnp.float32)
    hblk = jnp.maximum(hblk + b1_ref[0], 0.0)
    w2s = jnp.sum(w2_ref[0], axis=1, keepdims=True)
    part = jnp.dot(hblk, w2s, preferred_element_type=jnp.float32)

    @pl.when(ht == 0)
    def _():
        out_ref[...] = part + jnp.sum(b2_ref[0])

    @pl.when(ht != 0)
    def _():
        out_ref[...] += part


def _ffn(bufx, w1, b1r, w2, b2r):
    return pl.pallas_call(
        _ffn_body,
        grid=(E_, H_ // HT),
        in_specs=[
            pl.BlockSpec((C_, D_), lambda e, h: (e, 0)),
            pl.BlockSpec((1, D_, HT), lambda e, h: (e, 0, h)),
            pl.BlockSpec((1, 1, HT), lambda e, h: (e, 0, h)),
            pl.BlockSpec((1, HT, D_), lambda e, h: (e, h, 0)),
            pl.BlockSpec((1, 1, D_), lambda e, h: (e, 0, 0)),
        ],
        out_specs=pl.BlockSpec((C_, 1), lambda e, h: (e, 0)),
        out_shape=jax.ShapeDtypeStruct((NSLOT, 1), jnp.float32),
    )(bufx, w1, b1r, w2, b2r)


def _combine_body(slot_hbm, gmul_hbm, s1_hbm, z_hbm, sl_v, gm_v, val_v, z_v, sem):
    wid = lax.axis_index("c") * NS + lax.axis_index("s")
    base = wid * TPW
    pltpu.sync_copy(slot_hbm.at[pl.ds(base, TPW)], sl_v)
    pltpu.sync_copy(gmul_hbm.at[pl.ds(base, TPW)], gm_v)
    for v in range(TPW // 16):
        idx = sl_v[pl.ds(v * 16, 16)]
        pltpu.async_copy(s1_hbm.at[idx], val_v, sem).wait()
        z_v[pl.ds(v * 16, 16)] = val_v[...] * gm_v[pl.ds(v * 16, 16)]
    pltpu.sync_copy(z_v, z_hbm.at[pl.ds(base, TPW)])


def _combine(slot, gmul, s1):
    mesh = plsc.VectorSubcoreMesh(core_axis_name="c", subcore_axis_name="s")
    return pl.kernel(
        _combine_body,
        out_type=jax.ShapeDtypeStruct((T_,), jnp.float32),
        mesh=mesh,
        scratch_types=[
            pltpu.VMEM((TPW,), jnp.int32),
            pltpu.VMEM((TPW,), jnp.float32),
            pltpu.VMEM((16,), jnp.float32),
            pltpu.VMEM((TPW,), jnp.float32),
            pltpu.SemaphoreType.DMA,
        ],
    )(slot, gmul, s1)


def _lsm_body(z_ref, out_ref):
    z = z_ref[...]
    m = jnp.max(z, axis=1, keepdims=True)
    out_ref[...] = z - m - jnp.log(jnp.sum(jnp.exp(z - m), axis=1, keepdims=True))


def _lsm(z2):
    return pl.pallas_call(
        _lsm_body,
        out_shape=jax.ShapeDtypeStruct((B_, S_), jnp.float32),
    )(z2)


def kernel(input, Wg, W1, b1, W2, b2):
    xf = input.reshape(T_, D_)
    slot, gmul, dst = _gating(xf, Wg)
    bufx = _scatter(dst.reshape(T_), xf)
    s1 = _ffn(bufx, W1, b1.reshape(E_, 1, H_), W2, b2.reshape(E_, 1, D_))
    z = _combine(slot.reshape(T_), gmul.reshape(T_), s1.reshape(NSLOT))
    return _lsm(z.reshape(B_, S_))


# speedup vs baseline: 1.2473x; 1.0028x over previous
import functools

import jax
import jax.numpy as jnp
from jax import lax
from jax.experimental import pallas as pl
from jax.experimental.pallas import tpu as pltpu
from jax.experimental.pallas import tpu_sc as plsc

B_, S_, D_, H_, E_ = 2, 2048, 1024, 2048, 8
T_ = B_ * S_
C_ = 640
NSLOT = E_ * C_
NC, NS = 2, 16
NW = NC * NS
TPW = T_ // NW
RBUF = NSLOT + NW

TB = 512
HT = 256


def _gate_body(x_ref, wg_ref, slot_ref, gmul_ref, dst_ref, carry_ref):
    i = pl.program_id(0)
    x = x_ref[...]
    logits = jnp.dot(x, wg_ref[...], preferred_element_type=jnp.float32)
    lmax = jnp.max(logits, axis=1, keepdims=True)
    gval = 1.0 / jnp.sum(jnp.exp(logits - lmax), axis=1, keepdims=True)
    eids = lax.broadcasted_iota(jnp.int32, logits.shape, 1)
    eidx = jnp.min(jnp.where(logits >= lmax, eids, E_), axis=1, keepdims=True)

    @pl.when(i == 0)
    def _():
        carry_ref[...] = jnp.zeros_like(carry_ref)

    onehot = (eids == eidx).astype(jnp.float32)
    r = lax.broadcasted_iota(jnp.int32, (TB, TB), 0)
    c = lax.broadcasted_iota(jnp.int32, (TB, TB), 1)
    tri = (r >= c).astype(jnp.float32)
    cnt = jnp.dot(tri, onehot, preferred_element_type=jnp.float32)
    cnt = cnt + carry_ref[...]
    carry_ref[...] = cnt[TB - 1:TB, :]
    pos = jnp.sum(onehot * cnt, axis=1, keepdims=True).astype(jnp.int32) - 1
    valid = pos < C_
    slot = eidx * C_ + jnp.minimum(pos, C_ - 1)
    trow = lax.broadcasted_iota(jnp.int32, (TB, 1), 0) + i * TB
    dump = NSLOT + trow // TPW
    slot_ref[...] = slot
    gmul_ref[...] = jnp.where(valid, gval, 0.0)
    dst_ref[...] = jnp.where(valid, slot, dump)


def _gating(xf, wg):
    return pl.pallas_call(
        _gate_body,
        grid=(T_ // TB,),
        in_specs=[
            pl.BlockSpec((TB, D_), lambda i: (i, 0)),
            pl.BlockSpec((D_, E_), lambda i: (0, 0)),
        ],
        out_specs=[
            pl.BlockSpec((TB, 1), lambda i: (i, 0)),
            pl.BlockSpec((TB, 1), lambda i: (i, 0)),
            pl.BlockSpec((TB, 1), lambda i: (i, 0)),
        ],
        out_shape=[
            jax.ShapeDtypeStruct((T_, 1), jnp.int32),
            jax.ShapeDtypeStruct((T_, 1), jnp.float32),
            jax.ShapeDtypeStruct((T_, 1), jnp.int32),
        ],
        scratch_shapes=[pltpu.VMEM((1, E_), jnp.float32)],
    )(xf, wg)


def _scatter_body(dst_hbm, x_hbm, bufx_hbm, dst_v, rows_v, sem):
    wid = lax.axis_index("c") * NS + lax.axis_index("s")
    base = wid * TPW
    pltpu.sync_copy(dst_hbm.at[pl.ds(base, TPW)], dst_v)
    for v in range(TPW // 16):
        idx = dst_v[pl.ds(v * 16, 16)]
        pltpu.sync_copy(x_hbm.at[pl.ds(base + v * 16, 16)], rows_v)
        pltpu.async_copy(rows_v, bufx_hbm.at[idx], sem).wait()


def _scatter(dst, xf):
    mesh = plsc.VectorSubcoreMesh(core_axis_name="c", subcore_axis_name="s")
    return pl.kernel(
        _scatter_body,
        out_type=jax.ShapeDtypeStruct((RBUF, D_), jnp.float32),
        mesh=mesh,
        scratch_types=[
            pltpu.VMEM((TPW,), jnp.int32),
            pltpu.VMEM((16, D_), jnp.float32),
            pltpu.SemaphoreType.DMA,
        ],
    )(dst, xf)


def _ffn_body(bufx_ref, w1_ref, b1_ref, w2_ref, b2_ref, out_ref):
    ht = pl.program_id(1)
    xb = bufx_ref[...].astype(jnp.bfloat16)
    w1 = w1_ref[0].astype(jnp.bfloat16)
    hblk = jnp.dot(xb, w1, preferred_element_type=jnp.float32)
    hblk = jnp.maximum(hblk + b1_ref[0], 0.0)
    w2s = jnp.sum(w2_ref[0], axis=1, keepdims=True)
    part = jnp.dot(hblk, w2s, preferred_element_type=jnp.float32)

    @pl.when(ht == 0)
    def _():
        out_ref[...] = part + jnp.sum(b2_ref[0])

    @pl.when(ht != 0)
    def _():
        out_ref[...] += part


def _ffn(bufx, w1, b1r, w2, b2r):
    return pl.pallas_call(
        _ffn_body,
        grid=(E_, H_ // HT),
        in_specs=[
            pl.BlockSpec((C_, D_), lambda e, h: (e, 0)),
            pl.BlockSpec((1, D_, HT), lambda e, h: (e, 0, h)),
            pl.BlockSpec((1, 1, HT), lambda e, h: (e, 0, h)),
            pl.BlockSpec((1, HT, D_), lambda e, h: (e, h, 0)),
            pl.BlockSpec((1, 1, D_), lambda e, h: (e, 0, 0)),
        ],
        out_specs=pl.BlockSpec((C_, 1), lambda e, h: (e, 0)),
        out_shape=jax.ShapeDtypeStruct((NSLOT, 1), jnp.float32),
    )(bufx, w1, b1r, w2, b2r)


def _combine_body(slot_hbm, gmul_hbm, s1_hbm, z_hbm, sl_v, gm_v, val_v, z_v, sem):
    wid = lax.axis_index("c") * NS + lax.axis_index("s")
    base = wid * TPW
    pltpu.sync_copy(slot_hbm.at[pl.ds(base, TPW)], sl_v)
    pltpu.sync_copy(gmul_hbm.at[pl.ds(base, TPW)], gm_v)
    for v in range(TPW // 16):
        idx = sl_v[pl.ds(v * 16, 16)]
        pltpu.async_copy(s1_hbm.at[idx], val_v, sem).wait()
        z_v[pl.ds(v * 16, 16)] = val_v[...] * gm_v[pl.ds(v * 16, 16)]
    pltpu.sync_copy(z_v, z_hbm.at[pl.ds(base, TPW)])


def _combine(slot, gmul, s1):
    mesh = plsc.VectorSubcoreMesh(core_axis_name="c", subcore_axis_name="s")
    return pl.kernel(
        _combine_body,
        out_type=jax.ShapeDtypeStruct((T_,), jnp.float32),
        mesh=mesh,
        scratch_types=[
            pltpu.VMEM((TPW,), jnp.int32),
            pltpu.VMEM((TPW,), jnp.float32),
            pltpu.VMEM((16,), jnp.float32),
            pltpu.VMEM((TPW,), jnp.float32),
            pltpu.SemaphoreType.DMA,
        ],
    )(slot, gmul, s1)


def _lsm_body(z_ref, out_ref):
    z = z_ref[...]
    m = jnp.max(z, axis=1, keepdims=True)
    out_ref[...] = z - m - jnp.log(jnp.sum(jnp.exp(z - m), axis=1, keepdims=True))


def _lsm(z2):
    return pl.pallas_call(
        _lsm_body,
        out_shape=jax.ShapeDtypeStruct((B_, S_), jnp.float32),
    )(z2)


def kernel(input, Wg, W1, b1, W2, b2):
    xf = input.reshape(T_, D_)
    slot, gmul, dst = _gating(xf, Wg)
    bufx = _scatter(dst.reshape(T_), xf)
    s1 = _ffn(bufx, W1, b1.reshape(E_, 1, H_), W2, b2.reshape(E_, 1, D_))
    z = _combine(slot.reshape(T_), gmul.reshape(T_), s1.reshape(NSLOT))
    return _lsm(z.reshape(B_, S_))
